# baseline (speedup 1.0000x reference)
import functools

import jax
import jax.numpy as jnp
from jax import lax
from jax.experimental import pallas as pl
from jax.experimental.pallas import tpu as pltpu
from jax.experimental.pallas import tpu_sc as plsc

TEX = 1024
CH = 16
RES = 512
NPTS = RES * RES

NC, NS, L = 2, 16, 16
NW = NC * NS
BPW = NPTS // NW
CHUNK = 128
NCHUNK = BPW // CHUNK
NPAIR = NCHUNK // 2
NGRP = CHUNK // L

LEVELS = ((0, 1024), (1024, 512), (1536, 256), (1792, 128))
NTAP = 16

_mesh = plsc.VectorSubcoreMesh(
    core_axis_name="c", subcore_axis_name="s", num_cores=NC, num_subcores=NS
)
_params = pltpu.CompilerParams(
    use_tc_tiling_on_sc=False, needs_layout_passes=False
)


def _floorf(v):
    f = v.astype(jnp.int32).astype(jnp.float32)
    return jnp.where(f > v, f - 1.0, f)


@functools.partial(
    pl.kernel,
    compiler_params=_params,
    out_type=tuple(
        jax.ShapeDtypeStruct((w * w, CH), jnp.float32) for _, w in LEVELS
    ),
    mesh=_mesh,
    scratch_types=[
        pltpu.VMEM((2, CH, TEX), jnp.float32),
        pltpu.VMEM((2, TEX, CH), jnp.float32),
        pltpu.SemaphoreType.DMA,
        pltpu.SemaphoreType.DMA,
        pltpu.SemaphoreType.DMA,
        pltpu.SemaphoreType.DMA,
    ],
)
def _sc_repack(d_hbm, t0_hbm, t1_hbm, t2_hbm, t3_hbm, in_v, out_v,
               isemA, isemB, osemA, osemB):
    tables = (t0_hbm, t1_hbm, t2_hbm, t3_hbm)
    isems = (isemA, isemB)
    osems = (osemA, osemB)
    wid = lax.axis_index("s") * NC + lax.axis_index("c")
    iota = lax.iota(jnp.int32, L)

    def fire_in(yrow, yoff, w, b):
        for c in range(CH):
            pltpu.async_copy(
                d_hbm.at[c, yoff + yrow, pl.ds(0, w)],
                in_v.at[b, c, pl.ds(0, w)],
                isems[b],
            )

    def drain_in(yrow, yoff, w, b):
        for c in range(CH):
            pltpu.make_async_copy(
                d_hbm.at[c, yoff + yrow, pl.ds(0, w)],
                in_v.at[b, c, pl.ds(0, w)],
                isems[b],
            ).wait()

    def transpose(w, b):
        def xg_body(xg, _):
            xidx = iota + xg * L
            for c in range(CH):
                v = in_v[b, c, pl.ds(xg * L, L)]
                plsc.store_scatter(
                    out_v.at[b], [xidx, jnp.full((L,), c, jnp.int32)], v
                )
            return 0

        lax.fori_loop(0, w // L, xg_body, 0)

    def fire_out(yrow, w, table, b):
        pltpu.async_copy(
            out_v.at[b, pl.ds(0, w)], table.at[pl.ds(yrow * w, w)], osems[b]
        )

    def drain_out(yrow, w, table, b):
        pltpu.make_async_copy(
            out_v.at[b, pl.ds(0, w)], table.at[pl.ds(yrow * w, w)], osems[b]
        ).wait()

    for l, (yoff, w) in enumerate(LEVELS):
        rpw = w // NW
        y0 = wid * rpw
        table = tables[l]

        fire_in(y0, yoff, w, 0)

        def pair_body(j, _, y0=y0, yoff=yoff, w=w, table=table, rpw=rpw):
            ya = y0 + 2 * j
            fire_in(ya + 1, yoff, w, 1)
            drain_in(ya, yoff, w, 0)

            @pl.when(j > 0)
            def _():
                drain_out(0, w, table, 0)

            transpose(w, 0)
            fire_out(ya, w, table, 0)

            @pl.when(j < rpw // 2 - 1)
            def _():
                fire_in(ya + 2, yoff, w, 0)

            drain_in(ya + 1, yoff, w, 1)

            @pl.when(j > 0)
            def _():
                drain_out(0, w, table, 1)

            transpose(w, 1)
            fire_out(ya + 1, w, table, 1)
            return 0

        lax.fori_loop(0, rpw // 2, pair_body, 0)
        drain_out(0, w, table, 0)
        drain_out(0, w, table, 1)


@functools.partial(
    pl.kernel,
    compiler_params=_params,
    out_type=jax.ShapeDtypeStruct((CH, NPTS), jnp.float32),
    mesh=_mesh,
    scratch_types=[
        pltpu.VMEM((BPW,), jnp.float32),
        pltpu.VMEM((BPW,), jnp.float32),
        pltpu.VMEM((2, NTAP, CHUNK), jnp.int32),
        pltpu.VMEM((2, NTAP, CHUNK), jnp.float32),
        pltpu.VMEM((2, CH, CHUNK), jnp.float32),
        pltpu.SemaphoreType.DMA,
        pltpu.SemaphoreType.DMA,
        pltpu.SemaphoreType.DMA,
        pltpu.SemaphoreType.DMA,
    ] + [
        pltpu.VMEM((CHUNK, CH), jnp.float32)
        for _ in range(2 * NTAP)
    ],
)
def _sc_lookup(uv_hbm, t0_hbm, t1_hbm, t2_hbm, t3_hbm, out_hbm, x_v, y_v,
               idx_v, w_v, out_v, gsemA, gsemB, fsemA, fsemB, *rows_refs):
    tables = (t0_hbm, t1_hbm, t2_hbm, t3_hbm)
    wid = lax.axis_index("s") * NC + lax.axis_index("c")
    base = wid * BPW

    pltpu.sync_copy(uv_hbm.at[0, pl.ds(base, BPW)], x_v)
    pltpu.sync_copy(uv_hbm.at[1, pl.ds(base, BPW)], y_v)

    iota = lax.iota(jnp.int32, L)

    def calc(co, b):
        @plsc.parallel_loop(0, NGRP, unroll=2)
        def calc_body(g):
            sl = pl.ds(g * L, L)
            px = x_v[pl.ds(co + g * L, L)]
            py = y_v[pl.ds(co + g * L, L)]
            for l, (_, w) in enumerate(LEVELS):
                half = w * 0.5
                off = (w - 1) * 0.5
                ixf = px * half + off
                iyf = py * half + off
                fx0 = _floorf(ixf)
                fy0 = _floorf(iyf)
                wx1 = ixf - fx0
                wy1 = iyf - fy0
                wx0 = 1.0 - wx1
                wy0 = 1.0 - wy1
                ix0 = fx0.astype(jnp.int32)
                iy0 = fy0.astype(jnp.int32)
                ix0c = jnp.clip(ix0, 0, w - 1)
                ix1c = jnp.clip(ix0 + 1, 0, w - 1)
                iy0c = jnp.clip(iy0, 0, w - 1)
                iy1c = jnp.clip(iy0 + 1, 0, w - 1)
                r0 = iy0c * w
                r1 = iy1c * w
                t = 4 * l
                idx_v[b, t + 0, sl] = r0 + ix0c
                idx_v[b, t + 1, sl] = r0 + ix1c
                idx_v[b, t + 2, sl] = r1 + ix0c
                idx_v[b, t + 3, sl] = r1 + ix1c
                w_v[b, t + 0, sl] = wy0 * wx0
                w_v[b, t + 1, sl] = wy0 * wx1
                w_v[b, t + 2, sl] = wy1 * wx0
                w_v[b, t + 3, sl] = wy1 * wx1

    def fire(b, gsem):
        for t in range(NTAP):
            pltpu.async_copy(
                tables[t // 4].at[idx_v.at[b, t]],
                rows_refs[b * NTAP + t],
                gsem,
            )

    def drain_gathers(b, gsem):
        for t in range(NTAP):
            pltpu.make_async_copy(
                tables[t // 4].at[idx_v.at[b, t]],
                rows_refs[b * NTAP + t],
                gsem,
            ).wait()

    def combine(b):
        def comb_body(g, _):
            sl = pl.ds(g * L, L)
            pidx = iota + g * L
            ws = [w_v[b, t, sl] for t in range(NTAP)]
            for ch in range(CH):
                chs = jnp.full((L,), ch, jnp.int32)
                acc = [None, None, None, None]
                for t in range(NTAP):
                    v = plsc.load_gather(rows_refs[b * NTAP + t], [pidx, chs])
                    a = t % 4
                    acc[a] = v * ws[t] if acc[a] is None else acc[a] + v * ws[t]
                out_v[b, ch, sl] = (acc[0] + acc[1]) + (acc[2] + acc[3])
            return 0

        lax.fori_loop(0, NGRP, comb_body, 0)

    def fire_flush(co, b, fsem):
        pltpu.async_copy(
            out_v.at[b], out_hbm.at[:, pl.ds(base + co, CHUNK)], fsem
        )

    def drain_flush(b, fsem):
        pltpu.make_async_copy(
            out_v.at[b], out_hbm.at[:, pl.ds(base, CHUNK)], fsem
        ).wait()

    calc(0, 0)
    fire(0, gsemA)

    def pair_body(i, _):
        c0 = 2 * i
        co0 = c0 * CHUNK
        co1 = co0 + CHUNK
        calc(co1, 1)
        fire(1, gsemB)
        drain_gathers(0, gsemA)

        @pl.when(i > 0)
        def _():
            drain_flush(0, fsemA)

        combine(0)
        fire_flush(co0, 0, fsemA)

        @pl.when(i < NPAIR - 1)
        def _():
            calc(co0 + 2 * CHUNK, 0)
            fire(0, gsemA)

        drain_gathers(1, gsemB)

        @pl.when(i > 0)
        def _():
            drain_flush(1, fsemB)

        combine(1)
        fire_flush(co1, 1, fsemB)
        return 0

    lax.fori_loop(0, NPAIR, pair_body, 0)

    drain_flush(0, fsemA)
    drain_flush(1, fsemB)


def kernel(uv_inputs, data):
    d3 = data[0]
    tables = _sc_repack(d3)
    uv2 = uv_inputs.reshape(2, NPTS)
    out = _sc_lookup(uv2, *tables)
    return out.reshape(1, CH, RES, RES)


# speedup vs baseline: 1.0568x; 1.0568x over previous
import functools

import jax
import jax.numpy as jnp
from jax import lax
from jax.experimental import pallas as pl
from jax.experimental.pallas import tpu as pltpu
from jax.experimental.pallas import tpu_sc as plsc

TEX = 1024
CH = 16
RES = 512
NPTS = RES * RES

NC, NS, L = 2, 16, 16
NW = NC * NS
BPW = NPTS // NW
CHUNK = 128
NCHUNK = BPW // CHUNK
NPAIR = NCHUNK // 2
NGRP = CHUNK // L

LEVELS = ((0, 1024), (1024, 512), (1536, 256), (1792, 128))
NTAP = 16

_mesh = plsc.VectorSubcoreMesh(
    core_axis_name="c", subcore_axis_name="s", num_cores=NC, num_subcores=NS
)
_params = pltpu.CompilerParams(
    use_tc_tiling_on_sc=False, needs_layout_passes=False
)


def _floorf(v):
    f = v.astype(jnp.int32).astype(jnp.float32)
    return jnp.where(f > v, f - 1.0, f)


@functools.partial(
    pl.kernel,
    compiler_params=_params,
    out_type=tuple(
        jax.ShapeDtypeStruct((w * w, CH), jnp.float32) for _, w in LEVELS
    ),
    mesh=_mesh,
    scratch_types=[
        pltpu.VMEM((2, CH, TEX), jnp.float32),
        pltpu.VMEM((2, TEX, CH), jnp.float32),
        pltpu.SemaphoreType.DMA,
        pltpu.SemaphoreType.DMA,
        pltpu.SemaphoreType.DMA,
        pltpu.SemaphoreType.DMA,
    ],
)
def _sc_repack(d_hbm, t0_hbm, t1_hbm, t2_hbm, t3_hbm, in_v, out_v,
               isemA, isemB, osemA, osemB):
    tables = (t0_hbm, t1_hbm, t2_hbm, t3_hbm)
    isems = (isemA, isemB)
    osems = (osemA, osemB)
    wid = lax.axis_index("s") * NC + lax.axis_index("c")
    iota = lax.iota(jnp.int32, L)

    def fire_in(yrow, yoff, w, b):
        for c in range(CH):
            pltpu.async_copy(
                d_hbm.at[c, yoff + yrow, pl.ds(0, w)],
                in_v.at[b, c, pl.ds(0, w)],
                isems[b],
            )

    def drain_in(yrow, yoff, w, b):
        for c in range(CH):
            pltpu.make_async_copy(
                d_hbm.at[c, yoff + yrow, pl.ds(0, w)],
                in_v.at[b, c, pl.ds(0, w)],
                isems[b],
            ).wait()

    def transpose(w, b):
        def xg_body(xg, _):
            xidx = iota + xg * L
            for c in range(CH):
                v = in_v[b, c, pl.ds(xg * L, L)]
                plsc.store_scatter(
                    out_v.at[b], [xidx, jnp.full((L,), c, jnp.int32)], v
                )
            return 0

        lax.fori_loop(0, w // L, xg_body, 0)

    def fire_out(yrow, w, table, b):
        pltpu.async_copy(
            out_v.at[b, pl.ds(0, w)], table.at[pl.ds(yrow * w, w)], osems[b]
        )

    def drain_out(yrow, w, table, b):
        pltpu.make_async_copy(
            out_v.at[b, pl.ds(0, w)], table.at[pl.ds(yrow * w, w)], osems[b]
        ).wait()

    for l, (yoff, w) in enumerate(LEVELS):
        rpw = w // NW
        y0 = wid * rpw
        table = tables[l]

        fire_in(y0, yoff, w, 0)

        def pair_body(j, _, y0=y0, yoff=yoff, w=w, table=table, rpw=rpw):
            ya = y0 + 2 * j
            fire_in(ya + 1, yoff, w, 1)
            drain_in(ya, yoff, w, 0)

            @pl.when(j > 0)
            def _():
                drain_out(0, w, table, 0)

            transpose(w, 0)
            fire_out(ya, w, table, 0)

            @pl.when(j < rpw // 2 - 1)
            def _():
                fire_in(ya + 2, yoff, w, 0)

            drain_in(ya + 1, yoff, w, 1)

            @pl.when(j > 0)
            def _():
                drain_out(0, w, table, 1)

            transpose(w, 1)
            fire_out(ya + 1, w, table, 1)
            return 0

        lax.fori_loop(0, rpw // 2, pair_body, 0)
        drain_out(0, w, table, 0)
        drain_out(0, w, table, 1)


@functools.partial(
    pl.kernel,
    compiler_params=_params,
    out_type=jax.ShapeDtypeStruct((CH, NPTS), jnp.float32),
    mesh=_mesh,
    scratch_types=[
        pltpu.VMEM((BPW,), jnp.float32),
        pltpu.VMEM((BPW,), jnp.float32),
        pltpu.VMEM((2, NTAP, CHUNK), jnp.int32),
        pltpu.VMEM((2, NTAP, CHUNK), jnp.float32),
        pltpu.VMEM((2, CH, CHUNK), jnp.float32),
        pltpu.SemaphoreType.DMA,
        pltpu.SemaphoreType.DMA,
        pltpu.SemaphoreType.DMA,
        pltpu.SemaphoreType.DMA,
    ] + [
        pltpu.VMEM((CHUNK, CH), jnp.float32)
        for _ in range(2 * NTAP)
    ],
)
def _sc_lookup(uv_hbm, t0_hbm, t1_hbm, t2_hbm, t3_hbm, out_hbm, x_v, y_v,
               idx_v, w_v, out_v, gsemA, gsemB, fsemA, fsemB, *rows_refs):
    tables = (t0_hbm, t1_hbm, t2_hbm, t3_hbm)
    wid = lax.axis_index("s") * NC + lax.axis_index("c")
    base = wid * BPW

    pltpu.sync_copy(uv_hbm.at[0, pl.ds(base, BPW)], x_v)
    pltpu.sync_copy(uv_hbm.at[1, pl.ds(base, BPW)], y_v)

    iota = lax.iota(jnp.int32, L)

    def calc(co, b):
        @plsc.parallel_loop(0, NGRP, unroll=2)
        def calc_body(g):
            sl = pl.ds(g * L, L)
            px = x_v[pl.ds(co + g * L, L)]
            py = y_v[pl.ds(co + g * L, L)]
            for l, (_, w) in enumerate(LEVELS):
                half = w * 0.5
                off = (w - 1) * 0.5
                ixf = px * half + off
                iyf = py * half + off
                fx0 = _floorf(ixf)
                fy0 = _floorf(iyf)
                wx1 = ixf - fx0
                wy1 = iyf - fy0
                wx0 = 1.0 - wx1
                wy0 = 1.0 - wy1
                ix0 = fx0.astype(jnp.int32)
                iy0 = fy0.astype(jnp.int32)
                ix0c = jnp.clip(ix0, 0, w - 1)
                ix1c = jnp.clip(ix0 + 1, 0, w - 1)
                iy0c = jnp.clip(iy0, 0, w - 1)
                iy1c = jnp.clip(iy0 + 1, 0, w - 1)
                r0 = iy0c * w
                r1 = iy1c * w
                t = 4 * l
                idx_v[b, t + 0, sl] = r0 + ix0c
                idx_v[b, t + 1, sl] = r0 + ix1c
                idx_v[b, t + 2, sl] = r1 + ix0c
                idx_v[b, t + 3, sl] = r1 + ix1c
                w_v[b, t + 0, sl] = wy0 * wx0
                w_v[b, t + 1, sl] = wy0 * wx1
                w_v[b, t + 2, sl] = wy1 * wx0
                w_v[b, t + 3, sl] = wy1 * wx1

    def fire(b, gsem):
        for t in range(NTAP):
            pltpu.async_copy(
                tables[t // 4].at[idx_v.at[b, t]],
                rows_refs[b * NTAP + t],
                gsem,
            )

    def drain_gathers(b, gsem):
        for t in range(NTAP):
            pltpu.make_async_copy(
                tables[t // 4].at[idx_v.at[b, t]],
                rows_refs[b * NTAP + t],
                gsem,
            ).wait()

    def combine(b):
        @plsc.parallel_loop(0, NGRP, unroll=2)
        def comb_body(g):
            sl = pl.ds(g * L, L)
            pidx = iota + g * L
            ws = [w_v[b, t, sl] for t in range(NTAP)]
            for ch in range(CH):
                chs = jnp.full((L,), ch, jnp.int32)
                acc = [None, None, None, None]
                for t in range(NTAP):
                    v = plsc.load_gather(rows_refs[b * NTAP + t], [pidx, chs])
                    a = t % 4
                    acc[a] = v * ws[t] if acc[a] is None else acc[a] + v * ws[t]
                out_v[b, ch, sl] = (acc[0] + acc[1]) + (acc[2] + acc[3])

    def fire_flush(co, b, fsem):
        pltpu.async_copy(
            out_v.at[b], out_hbm.at[:, pl.ds(base + co, CHUNK)], fsem
        )

    def drain_flush(b, fsem):
        pltpu.make_async_copy(
            out_v.at[b], out_hbm.at[:, pl.ds(base, CHUNK)], fsem
        ).wait()

    calc(0, 0)
    fire(0, gsemA)

    def pair_body(i, _):
        c0 = 2 * i
        co0 = c0 * CHUNK
        co1 = co0 + CHUNK
        calc(co1, 1)
        fire(1, gsemB)
        drain_gathers(0, gsemA)

        @pl.when(i > 0)
        def _():
            drain_flush(0, fsemA)

        combine(0)
        fire_flush(co0, 0, fsemA)

        @pl.when(i < NPAIR - 1)
        def _():
            calc(co0 + 2 * CHUNK, 0)
            fire(0, gsemA)

        drain_gathers(1, gsemB)

        @pl.when(i > 0)
        def _():
            drain_flush(1, fsemB)

        combine(1)
        fire_flush(co1, 1, fsemB)
        return 0

    lax.fori_loop(0, NPAIR, pair_body, 0)

    drain_flush(0, fsemA)
    drain_flush(1, fsemB)


def kernel(uv_inputs, data):
    d3 = data[0]
    tables = _sc_repack(d3)
    uv2 = uv_inputs.reshape(2, NPTS)
    out = _sc_lookup(uv2, *tables)
    return out.reshape(1, CH, RES, RES)


# speedup vs baseline: 1.0600x; 1.0030x over previous
import functools

import jax
import jax.numpy as jnp
from jax import lax
from jax.experimental import pallas as pl
from jax.experimental.pallas import tpu as pltpu
from jax.experimental.pallas import tpu_sc as plsc

TEX = 1024
CH = 16
RES = 512
NPTS = RES * RES

NC, NS, L = 2, 16, 16
NW = NC * NS
BPW = NPTS // NW
CHUNK = 128
NCHUNK = BPW // CHUNK
NPAIR = NCHUNK // 2
NGRP = CHUNK // L

LEVELS = ((0, 1024), (1024, 512), (1536, 256), (1792, 128))
NTAP = 16

_mesh = plsc.VectorSubcoreMesh(
    core_axis_name="c", subcore_axis_name="s", num_cores=NC, num_subcores=NS
)
_params = pltpu.CompilerParams(
    use_tc_tiling_on_sc=False, needs_layout_passes=False
)


def _floorf(v):
    f = v.astype(jnp.int32).astype(jnp.float32)
    return jnp.where(f > v, f - 1.0, f)


@functools.partial(
    pl.kernel,
    compiler_params=_params,
    out_type=tuple(
        jax.ShapeDtypeStruct((w * w, CH), jnp.float32) for _, w in LEVELS
    ),
    mesh=_mesh,
    scratch_types=[
        pltpu.VMEM((2, CH, TEX), jnp.float32),
        pltpu.VMEM((2, TEX, CH), jnp.float32),
        pltpu.SemaphoreType.DMA,
        pltpu.SemaphoreType.DMA,
        pltpu.SemaphoreType.DMA,
        pltpu.SemaphoreType.DMA,
    ],
)
def _sc_repack(d_hbm, t0_hbm, t1_hbm, t2_hbm, t3_hbm, in_v, out_v,
               isemA, isemB, osemA, osemB):
    tables = (t0_hbm, t1_hbm, t2_hbm, t3_hbm)
    isems = (isemA, isemB)
    osems = (osemA, osemB)
    wid = lax.axis_index("s") * NC + lax.axis_index("c")
    iota = lax.iota(jnp.int32, L)

    def fire_in(yrow, yoff, w, b):
        for c in range(CH):
            pltpu.async_copy(
                d_hbm.at[c, yoff + yrow, pl.ds(0, w)],
                in_v.at[b, c, pl.ds(0, w)],
                isems[b],
            )

    def drain_in(yrow, yoff, w, b):
        for c in range(CH):
            pltpu.make_async_copy(
                d_hbm.at[c, yoff + yrow, pl.ds(0, w)],
                in_v.at[b, c, pl.ds(0, w)],
                isems[b],
            ).wait()

    def transpose(w, b):
        @plsc.parallel_loop(0, w // L, unroll=2)
        def xg_body(xg):
            xidx = iota + xg * L
            for c in range(CH):
                v = in_v[b, c, pl.ds(xg * L, L)]
                plsc.store_scatter(
                    out_v.at[b], [xidx, jnp.full((L,), c, jnp.int32)], v
                )

    def fire_out(yrow, w, table, b):
        pltpu.async_copy(
            out_v.at[b, pl.ds(0, w)], table.at[pl.ds(yrow * w, w)], osems[b]
        )

    def drain_out(yrow, w, table, b):
        pltpu.make_async_copy(
            out_v.at[b, pl.ds(0, w)], table.at[pl.ds(yrow * w, w)], osems[b]
        ).wait()

    for l, (yoff, w) in enumerate(LEVELS):
        rpw = w // NW
        y0 = wid * rpw
        table = tables[l]

        fire_in(y0, yoff, w, 0)

        def pair_body(j, _, y0=y0, yoff=yoff, w=w, table=table, rpw=rpw):
            ya = y0 + 2 * j
            fire_in(ya + 1, yoff, w, 1)
            drain_in(ya, yoff, w, 0)

            @pl.when(j > 0)
            def _():
                drain_out(0, w, table, 0)

            transpose(w, 0)
            fire_out(ya, w, table, 0)

            @pl.when(j < rpw // 2 - 1)
            def _():
                fire_in(ya + 2, yoff, w, 0)

            drain_in(ya + 1, yoff, w, 1)

            @pl.when(j > 0)
            def _():
                drain_out(0, w, table, 1)

            transpose(w, 1)
            fire_out(ya + 1, w, table, 1)
            return 0

        lax.fori_loop(0, rpw // 2, pair_body, 0)
        drain_out(0, w, table, 0)
        drain_out(0, w, table, 1)


@functools.partial(
    pl.kernel,
    compiler_params=_params,
    out_type=jax.ShapeDtypeStruct((CH, NPTS), jnp.float32),
    mesh=_mesh,
    scratch_types=[
        pltpu.VMEM((BPW,), jnp.float32),
        pltpu.VMEM((BPW,), jnp.float32),
        pltpu.VMEM((2, NTAP, CHUNK), jnp.int32),
        pltpu.VMEM((2, NTAP, CHUNK), jnp.float32),
        pltpu.VMEM((2, CH, CHUNK), jnp.float32),
        pltpu.SemaphoreType.DMA,
        pltpu.SemaphoreType.DMA,
        pltpu.SemaphoreType.DMA,
        pltpu.SemaphoreType.DMA,
    ] + [
        pltpu.VMEM((CHUNK, CH), jnp.float32)
        for _ in range(2 * NTAP)
    ],
)
def _sc_lookup(uv_hbm, t0_hbm, t1_hbm, t2_hbm, t3_hbm, out_hbm, x_v, y_v,
               idx_v, w_v, out_v, gsemA, gsemB, fsemA, fsemB, *rows_refs):
    tables = (t0_hbm, t1_hbm, t2_hbm, t3_hbm)
    wid = lax.axis_index("s") * NC + lax.axis_index("c")
    base = wid * BPW

    pltpu.sync_copy(uv_hbm.at[0, pl.ds(base, BPW)], x_v)
    pltpu.sync_copy(uv_hbm.at[1, pl.ds(base, BPW)], y_v)

    iota = lax.iota(jnp.int32, L)

    def calc(co, b):
        @plsc.parallel_loop(0, NGRP, unroll=2)
        def calc_body(g):
            sl = pl.ds(g * L, L)
            px = x_v[pl.ds(co + g * L, L)]
            py = y_v[pl.ds(co + g * L, L)]
            for l, (_, w) in enumerate(LEVELS):
                half = w * 0.5
                off = (w - 1) * 0.5
                ixf = px * half + off
                iyf = py * half + off
                fx0 = _floorf(ixf)
                fy0 = _floorf(iyf)
                wx1 = ixf - fx0
                wy1 = iyf - fy0
                wx0 = 1.0 - wx1
                wy0 = 1.0 - wy1
                ix0 = fx0.astype(jnp.int32)
                iy0 = fy0.astype(jnp.int32)
                ix0c = jnp.clip(ix0, 0, w - 1)
                ix1c = jnp.clip(ix0 + 1, 0, w - 1)
                iy0c = jnp.clip(iy0, 0, w - 1)
                iy1c = jnp.clip(iy0 + 1, 0, w - 1)
                r0 = iy0c * w
                r1 = iy1c * w
                t = 4 * l
                idx_v[b, t + 0, sl] = r0 + ix0c
                idx_v[b, t + 1, sl] = r0 + ix1c
                idx_v[b, t + 2, sl] = r1 + ix0c
                idx_v[b, t + 3, sl] = r1 + ix1c
                w_v[b, t + 0, sl] = wy0 * wx0
                w_v[b, t + 1, sl] = wy0 * wx1
                w_v[b, t + 2, sl] = wy1 * wx0
                w_v[b, t + 3, sl] = wy1 * wx1

    def fire(b, gsem):
        for t in range(NTAP):
            pltpu.async_copy(
                tables[t // 4].at[idx_v.at[b, t]],
                rows_refs[b * NTAP + t],
                gsem,
            )

    def drain_gathers(b, gsem):
        for t in range(NTAP):
            pltpu.make_async_copy(
                tables[t // 4].at[idx_v.at[b, t]],
                rows_refs[b * NTAP + t],
                gsem,
            ).wait()

    def combine(b):
        @plsc.parallel_loop(0, NGRP, unroll=2)
        def comb_body(g):
            sl = pl.ds(g * L, L)
            pidx = iota + g * L
            ws = [w_v[b, t, sl] for t in range(NTAP)]
            for ch in range(CH):
                chs = jnp.full((L,), ch, jnp.int32)
                acc = [None, None, None, None]
                for t in range(NTAP):
                    v = plsc.load_gather(rows_refs[b * NTAP + t], [pidx, chs])
                    a = t % 4
                    acc[a] = v * ws[t] if acc[a] is None else acc[a] + v * ws[t]
                out_v[b, ch, sl] = (acc[0] + acc[1]) + (acc[2] + acc[3])

    def fire_flush(co, b, fsem):
        pltpu.async_copy(
            out_v.at[b], out_hbm.at[:, pl.ds(base + co, CHUNK)], fsem
        )

    def drain_flush(b, fsem):
        pltpu.make_async_copy(
            out_v.at[b], out_hbm.at[:, pl.ds(base, CHUNK)], fsem
        ).wait()

    calc(0, 0)
    fire(0, gsemA)

    def pair_body(i, _):
        c0 = 2 * i
        co0 = c0 * CHUNK
        co1 = co0 + CHUNK
        calc(co1, 1)
        fire(1, gsemB)
        drain_gathers(0, gsemA)

        @pl.when(i > 0)
        def _():
            drain_flush(0, fsemA)

        combine(0)
        fire_flush(co0, 0, fsemA)

        @pl.when(i < NPAIR - 1)
        def _():
            calc(co0 + 2 * CHUNK, 0)
            fire(0, gsemA)

        drain_gathers(1, gsemB)

        @pl.when(i > 0)
        def _():
            drain_flush(1, fsemB)

        combine(1)
        fire_flush(co1, 1, fsemB)
        return 0

    lax.fori_loop(0, NPAIR, pair_body, 0)

    drain_flush(0, fsemA)
    drain_flush(1, fsemB)


def kernel(uv_inputs, data):
    d3 = data[0]
    tables = _sc_repack(d3)
    uv2 = uv_inputs.reshape(2, NPTS)
    out = _sc_lookup(uv2, *tables)
    return out.reshape(1, CH, RES, RES)
